# baseline (speedup 1.0000x reference)
import functools

import jax
import jax.numpy as jnp
from jax import lax
from jax.experimental import pallas as pl
from jax.experimental.pallas import tpu as pltpu
from jax.experimental.pallas import tpu_sc as plsc

_BATCH = 2
_SEQ = 8192
_EMBED = 1024

_NC = 2
_NS = 16
_NW = _NC * _NS
_L = 16

_FLOATS_PER_W = _SEQ * _EMBED // _NW
_CH = 32768
_NCHUNK = _FLOATS_PER_W // _CH


def _sc_add_body(x_hbm, pos_hbm, out_hbm, pos_buf, x_buf):
    wid = lax.axis_index("s") * _NC + lax.axis_index("c")
    base = wid * _FLOATS_PER_W
    for ci in range(_NCHUNK):
        coff = base + ci * _CH
        pltpu.sync_copy(pos_hbm.at[pl.ds(coff, _CH)], pos_buf)
        for b in range(_BATCH):
            xoff = b * (_SEQ * _EMBED) + coff
            pltpu.sync_copy(x_hbm.at[pl.ds(xoff, _CH)], x_buf)

            @plsc.parallel_loop(0, _CH, step=_L, unroll=8)
            def _(i):
                x_buf[pl.ds(i, _L)] = x_buf[pl.ds(i, _L)] + pos_buf[pl.ds(i, _L)]

            pltpu.sync_copy(x_buf, out_hbm.at[pl.ds(xoff, _CH)])


_sc_add = pl.kernel(
    _sc_add_body,
    out_type=jax.ShapeDtypeStruct((_BATCH * _SEQ * _EMBED,), jnp.float32),
    mesh=plsc.VectorSubcoreMesh(core_axis_name="c", subcore_axis_name="s"),
    scratch_types=[
        pltpu.VMEM((_CH,), jnp.float32),
        pltpu.VMEM((_CH,), jnp.float32),
    ],
)


def kernel(x, position_matrix):
    out_flat = _sc_add(x.reshape(-1), position_matrix.reshape(-1))
    return out_flat.reshape(x.shape)


# speedup vs baseline: 1.1972x; 1.1972x over previous
import jax
import jax.numpy as jnp
from jax import lax
from jax.experimental import pallas as pl
from jax.experimental.pallas import tpu as pltpu
from jax.experimental.pallas import tpu_sc as plsc

_BATCH = 2
_SEQ = 8192
_EMBED = 1024

_NC = 2
_NS = 16
_NW = _NC * _NS
_L = 16

_FLOATS_PER_W = _SEQ * _EMBED // _NW
_CH = 16384
_NCHUNK = _FLOATS_PER_W // _CH
_ITEMS = _NCHUNK * _BATCH


def _sc_add_body(x_hbm, pos_hbm, out_hbm,
                 xin0, xin1, yout0, yout1, pbuf0, pbuf1,
                 xs0, xs1, ys0, ys1, ps0, ps1):
    xin = (xin0, xin1)
    yout = (yout0, yout1)
    pbuf = (pbuf0, pbuf1)
    xs = (xs0, xs1)
    ys = (ys0, ys1)
    ps = (ps0, ps1)

    wid = lax.axis_index("s") * _NC + lax.axis_index("c")
    base = wid * _FLOATS_PER_W

    def x_slice(k):
        ci, b = divmod(k, _BATCH)
        off = b * (_SEQ * _EMBED) + base + ci * _CH
        return pl.ds(off, _CH)

    def pos_slice(ci):
        return pl.ds(base + ci * _CH, _CH)

    def x_load(k):
        return pltpu.make_async_copy(x_hbm.at[x_slice(k)], xin[k % 2], xs[k % 2])

    def pos_load(ci):
        return pltpu.make_async_copy(pos_hbm.at[pos_slice(ci)], pbuf[ci % 2],
                                     ps[ci % 2])

    def store(k):
        return pltpu.make_async_copy(yout[k % 2], out_hbm.at[x_slice(k)],
                                     ys[k % 2])

    pos_load(0).start()
    x_load(0).start()
    x_load(1).start()
    pos_load(1).start()

    for k in range(_ITEMS):
        ci, b = divmod(k, _BATCH)
        x_load(k).wait()
        if b == 0:
            pos_load(ci).wait()
        if k >= 2:
            store(k - 2).wait()

        src = xin[k % 2]
        dst = yout[k % 2]
        pb = pbuf[ci % 2]

        @plsc.parallel_loop(0, _CH, step=_L, unroll=8)
        def _(i):
            dst[pl.ds(i, _L)] = src[pl.ds(i, _L)] + pb[pl.ds(i, _L)]

        store(k).start()
        if k + 2 < _ITEMS:
            x_load(k + 2).start()
        if b == 1 and ci + 2 < _NCHUNK:
            pos_load(ci + 2).start()

    store(_ITEMS - 2).wait()
    store(_ITEMS - 1).wait()


_sc_add = pl.kernel(
    _sc_add_body,
    out_type=jax.ShapeDtypeStruct((_BATCH * _SEQ * _EMBED,), jnp.float32),
    mesh=plsc.VectorSubcoreMesh(core_axis_name="c", subcore_axis_name="s"),
    scratch_types=[
        pltpu.VMEM((_CH,), jnp.float32),
        pltpu.VMEM((_CH,), jnp.float32),
        pltpu.VMEM((_CH,), jnp.float32),
        pltpu.VMEM((_CH,), jnp.float32),
        pltpu.VMEM((_CH,), jnp.float32),
        pltpu.VMEM((_CH,), jnp.float32),
        pltpu.SemaphoreType.DMA,
        pltpu.SemaphoreType.DMA,
        pltpu.SemaphoreType.DMA,
        pltpu.SemaphoreType.DMA,
        pltpu.SemaphoreType.DMA,
        pltpu.SemaphoreType.DMA,
    ],
)


def kernel(x, position_matrix):
    out_flat = _sc_add(x.reshape(-1), position_matrix.reshape(-1))
    return out_flat.reshape(x.shape)


# speedup vs baseline: 3.2398x; 2.7062x over previous
import jax
import jax.numpy as jnp
from jax import lax
from jax.experimental import pallas as pl
from jax.experimental.pallas import tpu as pltpu
from jax.experimental.pallas import tpu_sc as plsc

_BATCH = 2
_SEQ = 8192
_EMBED = 1024

_NC = 2
_NS = 16
_NW = _NC * _NS
_L = 16

_ROWS_PER_W = _SEQ // _NW
_R = 8
_NCHUNK = _ROWS_PER_W // _R


def _sc_add_body(x_hbm, pos_hbm, out_hbm,
                 x0a, x0b, x1a, x1b, y0a, y0b, y1a, y1b, pba, pbb,
                 sx0a, sx0b, sx1a, sx1b, sy0a, sy0b, sy1a, sy1b, spa, spb):
    x0 = (x0a, x0b)
    x1 = (x1a, x1b)
    y0 = (y0a, y0b)
    y1 = (y1a, y1b)
    pb = (pba, pbb)
    sx0 = (sx0a, sx0b)
    sx1 = (sx1a, sx1b)
    sy0 = (sy0a, sy0b)
    sy1 = (sy1a, sy1b)
    sp = (spa, spb)

    wid = lax.axis_index("s") * _NC + lax.axis_index("c")
    row_base = wid * _ROWS_PER_W

    def loads(ci, j):
        r0 = row_base + ci * _R
        return (
            pltpu.make_async_copy(pos_hbm.at[pl.ds(r0, _R), :], pb[j], sp[j]),
            pltpu.make_async_copy(x_hbm.at[pl.ds(r0, _R), :], x0[j], sx0[j]),
            pltpu.make_async_copy(x_hbm.at[pl.ds(_SEQ + r0, _R), :],
                                  x1[j], sx1[j]),
        )

    def stores(ci, j):
        r0 = row_base + ci * _R
        return (
            pltpu.make_async_copy(y0[j], out_hbm.at[pl.ds(r0, _R), :], sy0[j]),
            pltpu.make_async_copy(y1[j], out_hbm.at[pl.ds(_SEQ + r0, _R), :],
                                  sy1[j]),
        )

    for c in loads(0, 0):
        c.start()
    for c in loads(1, 1):
        c.start()

    def step(p, carry):
        for j in (0, 1):
            ci = 2 * p + j
            for c in loads(ci, j):
                c.wait()

            @pl.when(ci >= 2)
            def _():
                for c in stores(ci - 2, j):
                    c.wait()

            x0j, x1j, y0j, y1j, pbj = x0[j], x1[j], y0[j], y1[j], pb[j]

            @plsc.parallel_loop(0, _R, step=1, unroll=1)
            def _(r):
                for t in range(_EMBED // _L):
                    cs = pl.ds(t * _L, _L)
                    pv = pbj[r, cs]
                    y0j[r, cs] = x0j[r, cs] + pv
                    y1j[r, cs] = x1j[r, cs] + pv

            for c in stores(ci, j):
                c.start()

            @pl.when(ci + 2 < _NCHUNK)
            def _():
                for c in loads(ci + 2, j):
                    c.start()
        return carry

    lax.fori_loop(0, _NCHUNK // 2, step, 0)

    for c in stores(_NCHUNK - 2, 0):
        c.wait()
    for c in stores(_NCHUNK - 1, 1):
        c.wait()


_sc_add = pl.kernel(
    _sc_add_body,
    out_type=jax.ShapeDtypeStruct((_BATCH * _SEQ, _EMBED), jnp.float32),
    mesh=plsc.VectorSubcoreMesh(core_axis_name="c", subcore_axis_name="s"),
    compiler_params=pltpu.CompilerParams(use_tc_tiling_on_sc=True),
    scratch_types=(
        [pltpu.VMEM((_R, _EMBED), jnp.float32)] * 10
        + [pltpu.SemaphoreType.DMA] * 10
    ),
)


def kernel(x, position_matrix):
    out2d = _sc_add(x.reshape(_BATCH * _SEQ, _EMBED), position_matrix)
    return out2d.reshape(x.shape)
